# Initial kernel scaffold; baseline (speedup 1.0000x reference)
import jax
import jax.numpy as jnp
from jax.experimental import pallas as pl


def kernel(input, w1s, w2s):
    raise NotImplementedError("write your pallas kernel here")


# speedup vs baseline: 6.6450x; 6.6450x over previous
import functools

import jax
import jax.numpy as jnp
from jax.experimental import pallas as pl
from jax.experimental.pallas import tpu as pltpu

NIN = 4096
NOUT = 4096
DEPTH = 12
N_NODES = 2**DEPTH - 1
NPAD = 4096


def _matmul_body(a_ref, b_ref, o_ref):
    @pl.when(pl.program_id(2) == 0)
    def _():
        o_ref[...] = jnp.zeros_like(o_ref)

    o_ref[...] += jnp.dot(a_ref[...], b_ref[...],
                          preferred_element_type=jnp.float32)


def _matmul(a, b, bm=512, bn=1024, bk=1024):
    m, k = a.shape
    _, n = b.shape
    return pl.pallas_call(
        _matmul_body,
        grid=(m // bm, n // bn, k // bk),
        in_specs=[
            pl.BlockSpec((bm, bk), lambda i, j, h: (i, h)),
            pl.BlockSpec((bk, bn), lambda i, j, h: (h, j)),
        ],
        out_specs=pl.BlockSpec((bm, bn), lambda i, j, h: (i, j)),
        out_shape=jax.ShapeDtypeStruct((m, n), jnp.float32),
        compiler_params=pltpu.CompilerParams(
            dimension_semantics=("parallel", "parallel", "arbitrary")),
    )(a, b)


def _walk_body(l_ref, a_ref):
    br = l_ref.shape[0]
    logits = l_ref[...]
    lane = jax.lax.broadcasted_iota(jnp.int32, (br, NPAD), 1)
    cur = jnp.zeros((br, 1), jnp.int32)
    acc = jnp.zeros((br, NPAD), jnp.float32)
    for _ in range(DEPTH):
        onehot = lane == cur
        sel = jnp.sum(jnp.where(onehot, logits, 0.0), axis=1, keepdims=True)
        act = jax.nn.gelu(sel)
        acc = acc + jnp.where(onehot, act, 0.0)
        cur = 2 * cur + 1 + (sel > 0).astype(jnp.int32)
    a_ref[...] = acc.astype(jnp.bfloat16)


def _walk(l, br=256):
    b = l.shape[0]
    return pl.pallas_call(
        _walk_body,
        grid=(b // br,),
        in_specs=[pl.BlockSpec((br, NPAD), lambda i: (i, 0))],
        out_specs=pl.BlockSpec((br, NPAD), lambda i: (i, 0)),
        out_shape=jax.ShapeDtypeStruct((b, NPAD), jnp.bfloat16),
        compiler_params=pltpu.CompilerParams(
            dimension_semantics=("parallel",)),
    )(l)


@jax.jit
def kernel(input, w1s, w2s):
    x = input
    pad = jnp.zeros((1, NIN), jnp.bfloat16)
    w1t = jnp.concatenate([w1s, pad], axis=0).T
    w2p = jnp.concatenate([w2s, pad], axis=0)
    logits = _matmul(x, w1t)
    acts = _walk(logits)
    y = _matmul(acts, w2p)
    return y.astype(jnp.bfloat16)


# speedup vs baseline: 9.5207x; 1.4328x over previous
import jax
import jax.numpy as jnp
from jax.experimental import pallas as pl
from jax.experimental.pallas import tpu as pltpu

NIN = 4096
NOUT = 4096
DEPTH = 12
N_NODES = 2**DEPTH - 1
NPAD = 4096
B = 4096

_DN_NT = (((1,), (1,)), ((), ()))


def _mm1_body(x_ref, w_ref, o_ref):
    @pl.when(pl.program_id(2) == 0)
    def _():
        o_ref[...] = jnp.zeros_like(o_ref)

    o_ref[...] += jax.lax.dot_general(
        x_ref[...], w_ref[...], _DN_NT, preferred_element_type=jnp.float32)


def _mm1(x, w1s, bm=512, bn=1024, bk=1024):
    return pl.pallas_call(
        _mm1_body,
        grid=(B // bm, NPAD // bn, NIN // bk),
        in_specs=[
            pl.BlockSpec((bm, bk), lambda i, j, h: (i, h)),
            pl.BlockSpec((bn, bk), lambda i, j, h: (j, h)),
        ],
        out_specs=pl.BlockSpec((bm, bn), lambda i, j, h: (i, j)),
        out_shape=jax.ShapeDtypeStruct((B, NPAD), jnp.float32),
        compiler_params=pltpu.CompilerParams(
            dimension_semantics=("parallel", "parallel", "arbitrary")),
    )(x, w1s)


def _windows():
    wins = []
    for d in range(DEPTH):
        first, last = 2**d - 1, 2**(d + 1) - 2
        lo = (first // 128) * 128
        hi = min(NPAD, (last // 128 + 1) * 128)
        wins.append((lo, hi))
    return wins


_WINS = _windows()


def _walk_body(l_ref, a_ref):
    br = l_ref.shape[0]
    a_ref[...] = jnp.zeros_like(a_ref)
    cur = jnp.zeros((br, 1), jnp.int32)
    for d in range(DEPTH):
        lo, hi = _WINS[d]
        lw = l_ref[:, lo:hi]
        lane = lo + jax.lax.broadcasted_iota(jnp.int32, (br, hi - lo), 1)
        onehot = lane == cur
        sel = jnp.sum(jnp.where(onehot, lw, 0.0), axis=1, keepdims=True)
        act = jax.nn.gelu(sel)
        a_ref[:, lo:hi] += jnp.where(onehot, act, 0.0).astype(jnp.bfloat16)
        cur = 2 * cur + 1 + (sel > 0).astype(jnp.int32)


def _walk(l, br=256):
    return pl.pallas_call(
        _walk_body,
        grid=(B // br,),
        in_specs=[pl.BlockSpec((br, NPAD), lambda i: (i, 0))],
        out_specs=pl.BlockSpec((br, NPAD), lambda i: (i, 0)),
        out_shape=jax.ShapeDtypeStruct((B, NPAD), jnp.bfloat16),
        compiler_params=pltpu.CompilerParams(
            dimension_semantics=("parallel",)),
    )(l)


def _mm2_body(a_ref, w_ref, o_ref, *, nk, bk):
    @pl.when(pl.program_id(2) == 0)
    def _():
        o_ref[...] = jnp.zeros_like(o_ref)

    w = w_ref[...]
    last = pl.program_id(2) == nk - 1
    ri = jax.lax.broadcasted_iota(jnp.int32, w.shape, 0)
    w = jnp.where(last & (ri == bk - 1), jnp.bfloat16(0), w)
    o_ref[...] += jax.lax.dot_general(
        a_ref[...], w, (((1,), (0,)), ((), ())),
        preferred_element_type=jnp.float32)


def _mm2(a, w2s, bm=512, bn=1024, bk=1024):
    import functools
    return pl.pallas_call(
        functools.partial(_mm2_body, nk=NPAD // bk, bk=bk),
        grid=(B // bm, NOUT // bn, NPAD // bk),
        in_specs=[
            pl.BlockSpec((bm, bk), lambda i, j, h: (i, h)),
            pl.BlockSpec((bk, bn), lambda i, j, h: (h, j)),
        ],
        out_specs=pl.BlockSpec((bm, bn), lambda i, j, h: (i, j)),
        out_shape=jax.ShapeDtypeStruct((B, NOUT), jnp.float32),
        compiler_params=pltpu.CompilerParams(
            dimension_semantics=("parallel", "parallel", "arbitrary")),
    )(a, w2s)


@jax.jit
def kernel(input, w1s, w2s):
    logits = _mm1(input, w1s)
    acts = _walk(logits)
    y = _mm2(acts, w2s)
    return y.astype(jnp.bfloat16)


# speedup vs baseline: 13.8958x; 1.4595x over previous
import jax
import jax.numpy as jnp
from jax.experimental import pallas as pl
from jax.experimental.pallas import tpu as pltpu

NIN = 4096
NOUT = 4096
DEPTH = 12
N_NODES = 2**DEPTH - 1
NPAD = 4096
B = 4096

_DN_NT = (((1,), (1,)), ((), ()))


def _mm1_body(x_ref, w_ref, o_ref):
    o_ref[...] = jax.lax.dot_general(
        x_ref[...], w_ref[...], _DN_NT, preferred_element_type=jnp.float32)


def _mm1(x, w1s, bm=1024, bn=1024):
    return pl.pallas_call(
        _mm1_body,
        grid=(B // bm, NPAD // bn),
        in_specs=[
            pl.BlockSpec((bm, NIN), lambda i, j: (i, 0)),
            pl.BlockSpec((bn, NIN), lambda i, j: (j, 0)),
        ],
        out_specs=pl.BlockSpec((bm, bn), lambda i, j: (i, j)),
        out_shape=jax.ShapeDtypeStruct((B, NPAD), jnp.float32),
        compiler_params=pltpu.CompilerParams(
            dimension_semantics=("parallel", "parallel")),
    )(x, w1s)


def _windows():
    wins = []
    for d in range(DEPTH):
        first, last = 2**d - 1, 2**(d + 1) - 2
        lo = (first // 128) * 128
        hi = min(NPAD, (last // 128 + 1) * 128)
        wins.append((lo, hi))
    return wins


_WINS = _windows()


def _walk_body(l_ref, a_ref):
    br = l_ref.shape[0]
    a_ref[...] = jnp.zeros_like(a_ref)
    cur = jnp.zeros((br, 1), jnp.int32)
    for d in range(DEPTH):
        lo, hi = _WINS[d]
        lw = l_ref[:, lo:hi]
        lane = lo + jax.lax.broadcasted_iota(jnp.int32, (br, hi - lo), 1)
        onehot = lane == cur
        sel = jnp.sum(jnp.where(onehot, lw, 0.0), axis=1, keepdims=True)
        act = jax.nn.gelu(sel)
        a_ref[:, lo:hi] += jnp.where(onehot, act, 0.0).astype(jnp.bfloat16)
        cur = 2 * cur + 1 + (sel > 0).astype(jnp.int32)


def _walk(l, br=256):
    return pl.pallas_call(
        _walk_body,
        grid=(B // br,),
        in_specs=[pl.BlockSpec((br, NPAD), lambda i: (i, 0))],
        out_specs=pl.BlockSpec((br, NPAD), lambda i: (i, 0)),
        out_shape=jax.ShapeDtypeStruct((B, NPAD), jnp.bfloat16),
        compiler_params=pltpu.CompilerParams(
            dimension_semantics=("parallel",)),
    )(l)


def _mm2_body(a_ref, w_ref, o_ref):
    w = w_ref[...]
    ri = jax.lax.broadcasted_iota(jnp.int32, w.shape, 0)
    w = jnp.where(ri == NPAD - 1, jnp.bfloat16(0), w)
    o_ref[...] = jax.lax.dot_general(
        a_ref[...], w, (((1,), (0,)), ((), ())),
        preferred_element_type=jnp.float32)


def _mm2(a, w2s, bm=1024, bn=1024):
    return pl.pallas_call(
        _mm2_body,
        grid=(B // bm, NOUT // bn),
        in_specs=[
            pl.BlockSpec((bm, NPAD), lambda i, j: (i, 0)),
            pl.BlockSpec((NPAD, bn), lambda i, j: (0, j)),
        ],
        out_specs=pl.BlockSpec((bm, bn), lambda i, j: (i, j)),
        out_shape=jax.ShapeDtypeStruct((B, NOUT), jnp.float32),
        compiler_params=pltpu.CompilerParams(
            dimension_semantics=("parallel", "parallel")),
    )(a, w2s)


@jax.jit
def kernel(input, w1s, w2s):
    logits = _mm1(input, w1s)
    acts = _walk(logits)
    y = _mm2(acts, w2s)
    return y.astype(jnp.bfloat16)
